# Initial kernel scaffold; baseline (speedup 1.0000x reference)
import jax
import jax.numpy as jnp
from jax.experimental import pallas as pl


def kernel(x, W):
    raise NotImplementedError("write your pallas kernel here")


# speedup vs baseline: 2.2532x; 2.2532x over previous
import functools

import jax
import jax.numpy as jnp
from jax.experimental import pallas as pl
from jax.experimental.pallas import tpu as pltpu

_RADIUS = 32


def _pe_kernel(w_ref, out_ref, strip_ref, *, L, RB, B):
    K, D = w_ref.shape
    pad = L - 1 - _RADIUS

    @pl.when(pl.program_id(0) == 0)
    def _build_strip():
        w = w_ref[...]
        strip_ref[pl.ds(0, pad), :] = jnp.broadcast_to(w[0:1, :], (pad, D))
        strip_ref[pl.ds(pad, K), :] = w
        strip_ref[pl.ds(pad + K, pad), :] = jnp.broadcast_to(w[K - 1:K, :], (pad, D))

    i0 = pl.program_id(0) * RB
    for r in range(RB):
        start = (L - 1) - (i0 + r)
        row = strip_ref[pl.ds(start, L), :]
        for b in range(B):
            out_ref[b, r, :, :] = row


def kernel(x, W):
    L, B = x.shape[0], x.shape[1]
    K, D = W.shape
    RB = 8
    out = pl.pallas_call(
        functools.partial(_pe_kernel, L=L, RB=RB, B=B),
        grid=(L // RB,),
        in_specs=[pl.BlockSpec((K, D), lambda i: (0, 0))],
        out_specs=pl.BlockSpec((B, RB, L, D), lambda i: (0, i, 0, 0)),
        out_shape=jax.ShapeDtypeStruct((B, L, L, D), jnp.float32),
        scratch_shapes=[pltpu.VMEM((2 * L, D), jnp.float32)],
    )(W)
    return out


# speedup vs baseline: 3.7768x; 1.6762x over previous
import functools

import jax
import jax.numpy as jnp
from jax.experimental import pallas as pl
from jax.experimental.pallas import tpu as pltpu

_RADIUS = 32


def _pe_kernel(w_ref, mid_ref, out_ref, strip_ref, *, L, RB, B, K):
    D = w_ref.shape[1]
    PH = 128 // D

    @pl.when(pl.program_id(0) == 0)
    def _build_strips():
        w = w_ref[...]
        n_edge = (L - 1 - _RADIUS + 1) // PH
        w0row = jnp.concatenate([w[0:1, :]] * PH, axis=1)
        wKrow = jnp.concatenate([w[K - 1:K, :]] * PH, axis=1)
        s0 = jnp.concatenate(
            [
                jnp.broadcast_to(w0row, (n_edge, 128)),
                mid_ref[...],
                jnp.broadcast_to(wKrow, (2 * L // PH - n_edge - (K - 1) // PH, 128)),
            ],
            axis=0,
        )
        roll1 = jnp.concatenate([s0[1:], s0[:1]], axis=0)
        strip_ref[0] = s0
        for k in range(1, PH):
            strip_ref[k] = jnp.concatenate(
                [s0[:, D * k:], roll1[:, : D * k]], axis=1
            )

    i0 = pl.program_id(0) * RB
    for r in range(RB):
        start = (L - 1) - (i0 + r)
        k = jax.lax.rem(start, PH)
        srow = jax.lax.div(start, PH)
        row = strip_ref[k, pl.ds(srow, L * D // 128), :]
        for b in range(B):
            out_ref[b, r] = row


def kernel(x, W):
    L, B = x.shape[0], x.shape[1]
    K, D = W.shape
    RB = 8
    PH = 128 // D
    W_mid = W[1:K].reshape((K - 1) * D // 128, 128)
    out = pl.pallas_call(
        functools.partial(_pe_kernel, L=L, RB=RB, B=B, K=K),
        grid=(L // RB,),
        in_specs=[
            pl.BlockSpec((K, D), lambda i: (0, 0)),
            pl.BlockSpec(W_mid.shape, lambda i: (0, 0)),
        ],
        out_specs=pl.BlockSpec((B, RB, L * D // 128, 128), lambda i: (0, i, 0, 0)),
        out_shape=jax.ShapeDtypeStruct((B, L, L * D // 128, 128), jnp.float32),
        scratch_shapes=[pltpu.VMEM((PH, 2 * L // PH, 128), jnp.float32)],
    )(W, W_mid)
    return out.reshape(B, L, L, D)


# speedup vs baseline: 3.8785x; 1.0269x over previous
import functools

import jax
import jax.numpy as jnp
from jax.experimental import pallas as pl
from jax.experimental.pallas import tpu as pltpu

_RADIUS = 32
_LAG = 32


def _pe_kernel(w_ref, mid_ref, out_ref, strip_ref, sem, *, L, B, K):
    D = w_ref.shape[1]
    PH = 128 // D
    SR = L * D // 128

    w = w_ref[...]
    n_edge = (L - 1 - _RADIUS + 1) // PH
    w0row = jnp.concatenate([w[0:1, :]] * PH, axis=1)
    wKrow = jnp.concatenate([w[K - 1:K, :]] * PH, axis=1)
    s0 = jnp.concatenate(
        [
            jnp.broadcast_to(w0row, (n_edge, 128)),
            mid_ref[...],
            jnp.broadcast_to(wKrow, (2 * L // PH - n_edge - (K - 1) // PH, 128)),
        ],
        axis=0,
    )
    roll1 = jnp.concatenate([s0[1:], s0[:1]], axis=0)
    strip_ref[0] = s0
    for k in range(1, PH):
        strip_ref[k] = jnp.concatenate([s0[:, D * k:], roll1[:, : D * k]], axis=1)

    def _copy(i, b):
        start = (L - 1) - i
        k = jax.lax.rem(start, PH)
        srow = jax.lax.div(start, PH)
        return pltpu.make_async_copy(
            strip_ref.at[k, pl.ds(srow, SR), :],
            out_ref.at[b, i],
            sem,
        )

    def _issue(i, carry):
        for b in range(B):
            _copy(i, b).start()

        @pl.when(i >= _LAG)
        def _drain():
            for b in range(B):
                _copy(i - _LAG, b).wait()

        return carry

    jax.lax.fori_loop(0, L, _issue, 0, unroll=2)

    def _final(i, carry):
        for b in range(B):
            _copy(L - _LAG + i, b).wait()
        return carry

    jax.lax.fori_loop(0, _LAG, _final, 0)


def kernel(x, W):
    L, B = x.shape[0], x.shape[1]
    K, D = W.shape
    PH = 128 // D
    W_mid = W[1:K].reshape((K - 1) * D // 128, 128)
    out = pl.pallas_call(
        functools.partial(_pe_kernel, L=L, B=B, K=K),
        in_specs=[
            pl.BlockSpec(memory_space=pltpu.MemorySpace.VMEM),
            pl.BlockSpec(memory_space=pltpu.MemorySpace.VMEM),
        ],
        out_specs=pl.BlockSpec(memory_space=pltpu.MemorySpace.HBM),
        out_shape=jax.ShapeDtypeStruct((B, L, L * D // 128, 128), jnp.float32),
        scratch_shapes=[
            pltpu.VMEM((PH, 2 * L // PH, 128), jnp.float32),
            pltpu.SemaphoreType.DMA,
        ],
    )(W, W_mid)
    return out.reshape(B, L, L, D)
